# baseline (speedup 1.0000x reference)
import functools

import jax
import jax.numpy as jnp
from jax import lax
from jax.experimental import pallas as pl
from jax.experimental.pallas import tpu as pltpu
from jax.experimental.pallas import tpu_sc as plsc

_D_IN = 768
_D_SAE = 8192
_T = 4
_K = 64
_PREFIX = (2048, 4096, 6144, 8192)
_B = 64
_DF = _T * _D_IN
_NSEG = 128
_SEG = _D_SAE // _NSEG
_NEG = -3.0e38


def _enc_body(x_ref, w_ref, b_ref, out_ref):
    out_ref[...] = (
        jnp.dot(x_ref[...], w_ref[...], preferred_element_type=jnp.float32)
        + b_ref[...]
    )


def _encode(x2, w2, b2):
    bs = 512
    return pl.pallas_call(
        _enc_body,
        grid=(_D_SAE // bs,),
        in_specs=[
            pl.BlockSpec((_B, _DF), lambda j: (0, 0)),
            pl.BlockSpec((_DF, bs), lambda j: (0, j)),
            pl.BlockSpec((1, bs), lambda j: (0, j)),
        ],
        out_specs=pl.BlockSpec((_B, bs), lambda j: (0, j)),
        out_shape=jax.ShapeDtypeStruct((_B, _D_SAE), jnp.float32),
    )(x2, w2, b2)


def _topk_mask_sc(pre):
    mesh = plsc.VectorSubcoreMesh(
        core_axis_name="c", subcore_axis_name="s", num_cores=2, num_subcores=16
    )

    @functools.partial(
        pl.kernel,
        out_type=jax.ShapeDtypeStruct((_B, _D_SAE), jnp.float32),
        mesh=mesh,
        compiler_params=pltpu.CompilerParams(needs_layout_passes=False),
        scratch_types=[
            pltpu.VMEM((_D_SAE,), jnp.float32),
            pltpu.VMEM((_D_SAE,), jnp.float32),
            pltpu.VMEM((_NSEG,), jnp.float32),
        ],
    )
    def topk(pre_h, z_h, orig, work, smref):
        wid = lax.axis_index("s") * 2 + lax.axis_index("c")
        lane = lax.iota(jnp.int32, 16)
        nvs = _SEG // 16
        nsm = _NSEG // 16
        for r in range(2):
            b = wid * 2 + r
            pltpu.sync_copy(pre_h.at[b], orig)
            pltpu.sync_copy(pre_h.at[b], work)

            sms = [jnp.full((16,), _NEG, jnp.float32) for _ in range(nsm)]
            for s in range(_NSEG):
                m = work[pl.ds(s * _SEG, 16)]
                for j in range(1, nvs):
                    m = jnp.maximum(m, work[pl.ds(s * _SEG + j * 16, 16)])
                ms = jnp.max(m)
                q, l = divmod(s, 16)
                sms[q] = jnp.where(lane == l, ms, sms[q])
            for q in range(nsm):
                smref[pl.ds(q * 16, 16)] = sms[q]

            def _it(i, tau):
                sv = [smref[pl.ds(q * 16, 16)] for q in range(nsm)]
                mm = sv[0]
                for q in range(1, nsm):
                    mm = jnp.maximum(mm, sv[q])
                m = jnp.max(mm)
                seg = jnp.int32(9999)
                for q in range(nsm):
                    eq = jnp.where(sv[q] == m, lane + q * 16, 9999)
                    seg = jnp.minimum(seg, jnp.min(eq).astype(jnp.int32))
                base = seg * _SEG

                done = jnp.int32(0)
                nmv = jnp.full((16,), _NEG, jnp.float32)
                for j in range(nvs):
                    v = work[pl.ds(base + j * 16, 16)]
                    pos = jnp.min(jnp.where(v == m, lane, 999)).astype(jnp.int32)
                    hit = (pos < 999) & (done == 0)
                    v2 = jnp.where((lane == pos) & hit, _NEG, v)
                    work[pl.ds(base + j * 16, 16)] = v2
                    nmv = jnp.maximum(nmv, v2)
                    done = jnp.where(hit, 1, done)
                nm = jnp.max(nmv)

                q16 = (seg // 16) * 16
                l = seg - q16
                sq = smref[pl.ds(q16, 16)]
                smref[pl.ds(q16, 16)] = jnp.where(lane == l, nm, sq)
                return m

            tau = lax.fori_loop(0, _K, _it, jnp.float32(0))

            def _emit(s, carry):
                v = orig[pl.ds(s * 16, 16)]
                z = jnp.where(
                    v >= tau, jnp.maximum(v, jnp.float32(0)), jnp.float32(0)
                )
                work[pl.ds(s * 16, 16)] = z
                return carry

            lax.fori_loop(0, _D_SAE // 16, _emit, 0)
            pltpu.sync_copy(work, z_h.at[b])

    return topk(pre)


def _dec_body(nk, want_xhat, z_ref, w_ref, x_ref, b_ref, loss_ref, xhat_ref,
              acc_ref):
    k = pl.program_id(0)

    @pl.when(k == 0)
    def _init():
        acc_ref[...] = jnp.zeros_like(acc_ref)

    zblk = z_ref[...]
    for t in range(_T):
        acc_ref[:, t, :] += jnp.dot(
            zblk, w_ref[:, t, :], preferred_element_type=jnp.float32
        )

    @pl.when(k == nk - 1)
    def _fin():
        xhat = acc_ref[...] + b_ref[...]
        if want_xhat:
            xhat_ref[...] = xhat
        d = xhat - x_ref[...]
        loss_ref[0, 0] = jnp.sum(d * d)


def _decode_scale(z, w, x3, b3, prefix, want_xhat):
    bk = 1024
    nk = prefix // bk
    outs = [jax.ShapeDtypeStruct((1, 1), jnp.float32)]
    out_specs = [pl.BlockSpec(memory_space=pltpu.SMEM)]
    if want_xhat:
        outs.append(jax.ShapeDtypeStruct((_B, _T, _D_IN), jnp.float32))
        out_specs.append(pl.BlockSpec((_B, _T, _D_IN), lambda k: (0, 0, 0)))
    else:
        outs.append(jax.ShapeDtypeStruct((1, 1, 1), jnp.float32))
        out_specs.append(pl.BlockSpec((1, 1, 1), lambda k: (0, 0, 0)))
    res = pl.pallas_call(
        functools.partial(_dec_body, nk, want_xhat),
        grid=(nk,),
        in_specs=[
            pl.BlockSpec((_B, bk), lambda k: (0, k)),
            pl.BlockSpec((bk, _T, _D_IN), lambda k: (k, 0, 0)),
            pl.BlockSpec((_B, _T, _D_IN), lambda k: (0, 0, 0)),
            pl.BlockSpec((1, _T, _D_IN), lambda k: (0, 0, 0)),
        ],
        out_specs=out_specs,
        out_shape=outs,
        scratch_shapes=[pltpu.VMEM((_B, _T, _D_IN), jnp.float32)],
    )(z, w, x3, b3)
    return res


def kernel(x, W_enc, b_enc, W_dec0, b_dec0, W_dec1, b_dec1, W_dec2, b_dec2,
           W_dec3, b_dec3):
    x2 = x.reshape(_B, _DF)
    w2 = W_enc.reshape(_DF, _D_SAE)
    pre = _encode(x2, w2, b_enc.reshape(1, _D_SAE))

    z = _topk_mask_sc(pre)

    wdecs = (W_dec0, W_dec1, W_dec2, W_dec3)
    bdecs = (b_dec0, b_dec1, b_dec2, b_dec3)
    total = jnp.zeros((), jnp.float32)
    last_xhat = None
    for i in range(4):
        res = _decode_scale(
            z, wdecs[i], x, bdecs[i].reshape(1, _T, _D_IN), _PREFIX[i],
            want_xhat=(i == 3),
        )
        total = total + res[0][0, 0]
        if i == 3:
            last_xhat = res[1]
    total = total / (4 * _B * _T)
    return (total, last_xhat, z)


# speedup vs baseline: 1.0096x; 1.0096x over previous
import functools

import jax
import jax.numpy as jnp
from jax import lax
from jax.experimental import pallas as pl
from jax.experimental.pallas import tpu as pltpu
from jax.experimental.pallas import tpu_sc as plsc

_D_IN = 768
_D_SAE = 8192
_T = 4
_K = 64
_PREFIX = (2048, 4096, 6144, 8192)
_B = 64
_DF = _T * _D_IN
_NSEG = 64
_SEG = _D_SAE // _NSEG
_NEG = -3.0e38


def _enc_body(x_ref, w_ref, b_ref, out_ref):
    out_ref[...] = (
        jnp.dot(x_ref[...], w_ref[...], preferred_element_type=jnp.float32)
        + b_ref[...]
    )


def _encode(x2, w2, b2):
    bs = 512
    return pl.pallas_call(
        _enc_body,
        grid=(_D_SAE // bs,),
        in_specs=[
            pl.BlockSpec((_B, _DF), lambda j: (0, 0)),
            pl.BlockSpec((_DF, bs), lambda j: (0, j)),
            pl.BlockSpec((1, bs), lambda j: (0, j)),
        ],
        out_specs=pl.BlockSpec((_B, bs), lambda j: (0, j)),
        out_shape=jax.ShapeDtypeStruct((_B, _D_SAE), jnp.float32),
    )(x2, w2, b2)


def _topk_mask_sc(pre):
    mesh = plsc.VectorSubcoreMesh(
        core_axis_name="c", subcore_axis_name="s", num_cores=2, num_subcores=16
    )

    @functools.partial(
        pl.kernel,
        out_type=jax.ShapeDtypeStruct((_B, _D_SAE), jnp.float32),
        mesh=mesh,
        compiler_params=pltpu.CompilerParams(needs_layout_passes=False),
        scratch_types=[
            pltpu.VMEM((_D_SAE,), jnp.float32),
            pltpu.VMEM((_D_SAE,), jnp.float32),
            pltpu.VMEM((_NSEG,), jnp.float32),
        ],
    )
    def topk(pre_h, z_h, orig, work, smref):
        wid = lax.axis_index("s") * 2 + lax.axis_index("c")
        lane = lax.iota(jnp.int32, 16)
        nvs = _SEG // 16
        nsm = _NSEG // 16
        for r in range(2):
            b = wid * 2 + r
            pltpu.sync_copy(pre_h.at[b], orig)
            pltpu.sync_copy(pre_h.at[b], work)

            sms = [jnp.full((16,), _NEG, jnp.float32) for _ in range(nsm)]
            for s in range(_NSEG):
                m = work[pl.ds(s * _SEG, 16)]
                for j in range(1, nvs):
                    m = jnp.maximum(m, work[pl.ds(s * _SEG + j * 16, 16)])
                ms = jnp.max(m)
                q, l = divmod(s, 16)
                sms[q] = jnp.where(lane == l, ms, sms[q])
            for q in range(nsm):
                smref[pl.ds(q * 16, 16)] = sms[q]

            def _it(i, tau):
                sv = [smref[pl.ds(q * 16, 16)] for q in range(nsm)]
                mm = sv[0]
                for q in range(1, nsm):
                    mm = jnp.maximum(mm, sv[q])
                m = jnp.max(mm)
                seg = jnp.int32(9999)
                for q in range(nsm):
                    eq = jnp.where(sv[q] == m, lane + q * 16, 9999)
                    seg = jnp.minimum(seg, jnp.min(eq).astype(jnp.int32))
                base = seg * _SEG

                done = jnp.int32(0)
                nmv = jnp.full((16,), _NEG, jnp.float32)
                for j in range(nvs):
                    v = work[pl.ds(base + j * 16, 16)]
                    pos = jnp.min(jnp.where(v == m, lane, 999)).astype(jnp.int32)
                    hit = (pos < 999) & (done == 0)
                    v2 = jnp.where((lane == pos) & hit, _NEG, v)
                    work[pl.ds(base + j * 16, 16)] = v2
                    nmv = jnp.maximum(nmv, v2)
                    done = jnp.where(hit, 1, done)
                nm = jnp.max(nmv)

                q16 = (seg // 16) * 16
                l = seg - q16
                sq = smref[pl.ds(q16, 16)]
                smref[pl.ds(q16, 16)] = jnp.where(lane == l, nm, sq)
                return m

            tau = lax.fori_loop(0, _K, _it, jnp.float32(0))

            def _emit(s, carry):
                v = orig[pl.ds(s * 16, 16)]
                z = jnp.where(
                    v >= tau, jnp.maximum(v, jnp.float32(0)), jnp.float32(0)
                )
                work[pl.ds(s * 16, 16)] = z
                return carry

            lax.fori_loop(0, _D_SAE // 16, _emit, 0)
            pltpu.sync_copy(work, z_h.at[b])

    return topk(pre)


def _dec_body(nk, want_xhat, z_ref, w_ref, x_ref, b_ref, loss_ref, xhat_ref,
              acc_ref):
    k = pl.program_id(0)

    @pl.when(k == 0)
    def _init():
        acc_ref[...] = jnp.zeros_like(acc_ref)

    zblk = z_ref[...]
    for t in range(_T):
        acc_ref[:, t, :] += jnp.dot(
            zblk, w_ref[:, t, :], preferred_element_type=jnp.float32
        )

    @pl.when(k == nk - 1)
    def _fin():
        xhat = acc_ref[...] + b_ref[...]
        if want_xhat:
            xhat_ref[...] = xhat
        d = xhat - x_ref[...]
        loss_ref[0, 0] = jnp.sum(d * d)


def _decode_scale(z, w, x3, b3, prefix, want_xhat):
    bk = min(512, prefix // 4) if prefix < 4096 else 1024
    nk = prefix // bk
    outs = [jax.ShapeDtypeStruct((1, 1), jnp.float32)]
    out_specs = [pl.BlockSpec(memory_space=pltpu.SMEM)]
    if want_xhat:
        outs.append(jax.ShapeDtypeStruct((_B, _T, _D_IN), jnp.float32))
        out_specs.append(pl.BlockSpec((_B, _T, _D_IN), lambda k: (0, 0, 0)))
    else:
        outs.append(jax.ShapeDtypeStruct((1, 1, 1), jnp.float32))
        out_specs.append(pl.BlockSpec((1, 1, 1), lambda k: (0, 0, 0)))
    res = pl.pallas_call(
        functools.partial(_dec_body, nk, want_xhat),
        grid=(nk,),
        in_specs=[
            pl.BlockSpec((_B, bk), lambda k: (0, k)),
            pl.BlockSpec((bk, _T, _D_IN), lambda k: (k, 0, 0)),
            pl.BlockSpec((_B, _T, _D_IN), lambda k: (0, 0, 0)),
            pl.BlockSpec((1, _T, _D_IN), lambda k: (0, 0, 0)),
        ],
        out_specs=out_specs,
        out_shape=outs,
        scratch_shapes=[pltpu.VMEM((_B, _T, _D_IN), jnp.float32)],
    )(z, w, x3, b3)
    return res


def kernel(x, W_enc, b_enc, W_dec0, b_dec0, W_dec1, b_dec1, W_dec2, b_dec2,
           W_dec3, b_dec3):
    x2 = x.reshape(_B, _DF)
    w2 = W_enc.reshape(_DF, _D_SAE)
    pre = _encode(x2, w2, b_enc.reshape(1, _D_SAE))

    z = _topk_mask_sc(pre)

    wdecs = (W_dec0, W_dec1, W_dec2, W_dec3)
    bdecs = (b_dec0, b_dec1, b_dec2, b_dec3)
    total = jnp.zeros((), jnp.float32)
    last_xhat = None
    for i in range(4):
        res = _decode_scale(
            z, wdecs[i], x, bdecs[i].reshape(1, _T, _D_IN), _PREFIX[i],
            want_xhat=(i == 3),
        )
        total = total + res[0][0, 0]
        if i == 3:
            last_xhat = res[1]
    total = total / (4 * _B * _T)
    return (total, last_xhat, z)
